# baseline (speedup 1.0000x reference)
import functools

import jax
import jax.numpy as jnp
from jax.experimental import pallas as pl
from jax.experimental.pallas import tpu as pltpu
from jax.experimental.pallas import tpu_sc as plsc

_SC_CORES = 2
_SC_SUBCORES = 16


def _copy_rot_body(pos_ref, knew_ref, cos_ref, sin_ref, ck_ref, cv_ref,
                   out_ref, rot_ref, cosb, sinb, sem):
    nb, u, h, d = rot_ref.shape
    i = pl.program_id(0)
    s = pl.program_id(1)

    @pl.when((i == 0) & (s == 0))
    def _prep():
        dmas = []
        for j in range(nb):
            bj = pos_ref[j, 0]
            gc = pltpu.make_async_copy(
                cos_ref.at[pl.ds(bj, u)], cosb.at[j], sem.at[0, j])
            gs = pltpu.make_async_copy(
                sin_ref.at[pl.ds(bj, u)], sinb.at[j], sem.at[1, j])
            gc.start()
            gs.start()
            dmas.append((gc, gs))
        for gc, gs in dmas:
            gc.wait()
            gs.wait()
        x = knew_ref[...]
        xp = pltpu.roll(x, d - 1, 3)
        xm = pltpu.roll(x, 1, 3)
        lane = jax.lax.broadcasted_iota(jnp.int32, x.shape, 3)
        even = (lane % 2) == 0
        half = d // 2
        er = jax.lax.broadcasted_iota(jnp.int32, (half, d), 0)
        ec = jax.lax.broadcasted_iota(jnp.int32, (half, d), 1)
        emat = (ec // 2 == er).astype(jnp.float32)
        cf = jnp.dot(cosb[...].reshape(nb * u, half), emat,
                     preferred_element_type=jnp.float32)
        cf = cf.reshape(nb, u, 1, d)
        sa = jnp.dot(sinb[...].reshape(nb * u, half), emat,
                     preferred_element_type=jnp.float32)
        sa = sa.reshape(nb, u, 1, d)
        sa = jnp.where(even[:, :, :1], -sa, sa)
        rot_ref[...] = x * cf + jnp.where(even, xp, xm) * sa

    out_ref[0, 0] = ck_ref[0]
    out_ref[1, 0] = cv_ref[0]


def _sc_scatter_body(out_ref, rot_ref, vnew_ref, scidx_ref,
                     rows_v, idx_v, sem):
    nw_half = _SC_CORES * _SC_SUBCORES // 2
    rows_per_w = rot_ref.shape[0] * 2 // (_SC_CORES * _SC_SUBCORES)
    cid = jax.lax.axis_index("c")
    sid = jax.lax.axis_index("s")
    w = sid * _SC_CORES + cid
    pltpu.sync_copy(scidx_ref.at[pl.ds(w * rows_per_w, rows_per_w)], idx_v)

    @pl.when(w < nw_half)
    def _stage_k():
        pltpu.sync_copy(rot_ref.at[pl.ds(w * rows_per_w, rows_per_w)], rows_v)

    @pl.when(w >= nw_half)
    def _stage_v():
        pltpu.sync_copy(
            vnew_ref.at[pl.ds((w - nw_half) * rows_per_w, rows_per_w)], rows_v)

    cp = pltpu.make_async_copy(rows_v, out_ref.at[idx_v], sem)
    cp.start()
    cp.wait()


@functools.partial(jax.jit, static_argnames=("interpret",))
def _run(k_new, v_new, cos, sin, cache_k, cache_v, positions, interpret=False):
    b, u, h, d = k_new.shape
    cl = cache_k.shape[1]
    hd = h * d
    half = d // 2
    f32 = jnp.float32

    t_blk = 512
    s_steps = cl // t_blk
    out1, rotb = pl.pallas_call(
        _copy_rot_body,
        grid=(b, s_steps),
        in_specs=[
            pl.BlockSpec(memory_space=pltpu.SMEM),
            pl.BlockSpec(memory_space=pltpu.VMEM),
            pl.BlockSpec(memory_space=pl.ANY),
            pl.BlockSpec(memory_space=pl.ANY),
            pl.BlockSpec((1, t_blk, h, d), lambda i, s: (i, s, 0, 0)),
            pl.BlockSpec((1, t_blk, h, d), lambda i, s: (i, s, 0, 0)),
        ],
        out_specs=[
            pl.BlockSpec((2, 1, t_blk, h, d), lambda i, s: (0, i, s, 0, 0)),
            pl.BlockSpec((b, u, h, d), lambda i, s: (0, 0, 0, 0)),
        ],
        out_shape=[
            jax.ShapeDtypeStruct((2, b, cl, h, d), f32),
            jax.ShapeDtypeStruct((b, u, h, d), f32),
        ],
        scratch_shapes=[
            pltpu.VMEM((b, u, 1, half), f32),
            pltpu.VMEM((b, u, 1, half), f32),
            pltpu.SemaphoreType.DMA((2, b)),
        ],
        interpret=interpret,
    )(positions, k_new, cos.reshape(cl, 1, half), sin.reshape(cl, 1, half),
      cache_k, cache_v)

    poff = jnp.arange(b, dtype=jnp.int32)[:, None] * cl + positions
    scidx = jnp.concatenate(
        [poff.reshape(-1), poff.reshape(-1) + b * cl])

    rows_per_w = 2 * b * u // (_SC_CORES * _SC_SUBCORES)
    mesh = plsc.VectorSubcoreMesh(
        core_axis_name="c", subcore_axis_name="s",
        num_cores=_SC_CORES, num_subcores=_SC_SUBCORES)
    sc_scatter = pl.kernel(
        _sc_scatter_body,
        out_type=(),
        mesh=mesh,
        scratch_types=[
            pltpu.VMEM((rows_per_w, hd), f32),
            pltpu.VMEM((rows_per_w,), jnp.int32),
            pltpu.SemaphoreType.DMA,
        ],
        interpret=interpret,
    )

    oref = jax.new_ref(out1.reshape(2 * b * cl, hd))
    sc_scatter(oref, rotb.reshape(b * u, hd), v_new.reshape(b * u, hd), scidx)
    return oref[...].reshape(2, b, cl, h, d)


def kernel(k_new, v_new, cos, sin, cache_k, cache_v, positions):
    return _run(k_new, v_new, cos, sin, cache_k, cache_v, positions)


# speedup vs baseline: 1.0008x; 1.0008x over previous
import functools

import jax
import jax.numpy as jnp
from jax.experimental import pallas as pl
from jax.experimental.pallas import tpu as pltpu
from jax.experimental.pallas import tpu_sc as plsc

_SC_CORES = 2
_SC_SUBCORES = 16


def _copy_rot_body(pos_ref, knew_ref, cos_ref, sin_ref, ck_ref, cv_ref,
                   out_ref, rot_ref, cosb, sinb, sem):
    nb, u, h, d = rot_ref.shape
    i = pl.program_id(0)
    s = pl.program_id(1)

    @pl.when((i == 0) & (s == 0))
    def _prep():
        dmas = []
        for j in range(nb):
            bj = pos_ref[j, 0]
            gc = pltpu.make_async_copy(
                cos_ref.at[pl.ds(bj, u)], cosb.at[j], sem.at[0, j])
            gs = pltpu.make_async_copy(
                sin_ref.at[pl.ds(bj, u)], sinb.at[j], sem.at[1, j])
            gc.start()
            gs.start()
            dmas.append((gc, gs))
        for gc, gs in dmas:
            gc.wait()
            gs.wait()
        x = knew_ref[...]
        xp = pltpu.roll(x, d - 1, 3)
        xm = pltpu.roll(x, 1, 3)
        lane = jax.lax.broadcasted_iota(jnp.int32, x.shape, 3)
        even = (lane % 2) == 0
        half = d // 2
        er = jax.lax.broadcasted_iota(jnp.int32, (half, d), 0)
        ec = jax.lax.broadcasted_iota(jnp.int32, (half, d), 1)
        emat = (ec // 2 == er).astype(jnp.float32)
        cf = jnp.dot(cosb[...].reshape(nb * u, half), emat,
                     preferred_element_type=jnp.float32)
        cf = cf.reshape(nb, u, 1, d)
        sa = jnp.dot(sinb[...].reshape(nb * u, half), emat,
                     preferred_element_type=jnp.float32)
        sa = sa.reshape(nb, u, 1, d)
        sa = jnp.where(even[:, :, :1], -sa, sa)
        rot_ref[...] = x * cf + jnp.where(even, xp, xm) * sa

    out_ref[0, 0] = ck_ref[0]
    out_ref[1, 0] = cv_ref[0]


def _sc_scatter_body(out_ref, rot_ref, vnew_ref, scidx_ref,
                     rows_v, idx_v, sem):
    nw_half = _SC_CORES * _SC_SUBCORES // 2
    rows_per_w = rot_ref.shape[0] * 2 // (_SC_CORES * _SC_SUBCORES)
    cid = jax.lax.axis_index("c")
    sid = jax.lax.axis_index("s")
    w = sid * _SC_CORES + cid
    pltpu.sync_copy(scidx_ref.at[pl.ds(w * rows_per_w, rows_per_w)], idx_v)

    @pl.when(w < nw_half)
    def _stage_k():
        pltpu.sync_copy(rot_ref.at[pl.ds(w * rows_per_w, rows_per_w)], rows_v)

    @pl.when(w >= nw_half)
    def _stage_v():
        pltpu.sync_copy(
            vnew_ref.at[pl.ds((w - nw_half) * rows_per_w, rows_per_w)], rows_v)

    cp = pltpu.make_async_copy(rows_v, out_ref.at[idx_v], sem)
    cp.start()
    cp.wait()


@functools.partial(jax.jit, static_argnames=("interpret",))
def _run(k_new, v_new, cos, sin, cache_k, cache_v, positions, interpret=False):
    b, u, h, d = k_new.shape
    cl = cache_k.shape[1]
    hd = h * d
    half = d // 2
    f32 = jnp.float32

    t_blk = 512
    s_steps = cl // t_blk
    out1, rotb = pl.pallas_call(
        _copy_rot_body,
        grid=(b, s_steps),
        in_specs=[
            pl.BlockSpec(memory_space=pltpu.SMEM),
            pl.BlockSpec(memory_space=pltpu.VMEM),
            pl.BlockSpec(memory_space=pl.ANY),
            pl.BlockSpec(memory_space=pl.ANY),
            pl.BlockSpec((1, t_blk, h, d), lambda i, s: (i, s, 0, 0)),
            pl.BlockSpec((1, t_blk, h, d), lambda i, s: (i, s, 0, 0)),
        ],
        out_specs=[
            pl.BlockSpec((2, 1, t_blk, h, d), lambda i, s: (0, i, s, 0, 0)),
            pl.BlockSpec((b, u, h, d), lambda i, s: (0, 0, 0, 0)),
        ],
        out_shape=[
            jax.ShapeDtypeStruct((2, b, cl, h, d), f32),
            jax.ShapeDtypeStruct((b, u, h, d), f32),
        ],
        scratch_shapes=[
            pltpu.VMEM((b, u, 1, half), f32),
            pltpu.VMEM((b, u, 1, half), f32),
            pltpu.SemaphoreType.DMA((2, b)),
        ],
        interpret=interpret,
    )(positions, k_new, cos.reshape(cl, 1, half), sin.reshape(cl, 1, half),
      cache_k, cache_v)

    poff = jnp.arange(b, dtype=jnp.int32)[:, None] * cl + positions
    scidx = jnp.concatenate(
        [poff.reshape(-1), poff.reshape(-1) + b * cl])

    rows_per_w = 2 * b * u // (_SC_CORES * _SC_SUBCORES)
    mesh = plsc.VectorSubcoreMesh(
        core_axis_name="c", subcore_axis_name="s",
        num_cores=_SC_CORES, num_subcores=_SC_SUBCORES)
    sc_scatter = pl.kernel(
        _sc_scatter_body,
        out_type=(),
        mesh=mesh,
        scratch_types=[
            pltpu.VMEM((rows_per_w, hd), f32),
            pltpu.VMEM((rows_per_w,), jnp.int32),
            pltpu.SemaphoreType.DMA,
        ],
        interpret=interpret,
    )

    oref = jax.new_ref(out1.reshape(2 * b * cl, hd))
    sc_scatter(oref, rotb.reshape(b * u, hd), v_new.reshape(b * u, hd), scidx)
    return jax.freeze(oref).reshape(2, b, cl, h, d)


def kernel(k_new, v_new, cos, sin, cache_k, cache_v, positions):
    return _run(k_new, v_new, cos, sin, cache_k, cache_v, positions)


# speedup vs baseline: 2.9143x; 2.9120x over previous
import functools

import jax
import jax.numpy as jnp
from jax.experimental import pallas as pl
from jax.experimental.pallas import tpu as pltpu
from jax.experimental.pallas import tpu_sc as plsc

_SC_CORES = 2
_SC_SUBCORES = 16


def _copy_rot_body(pos_ref, knew_ref, cos_ref, sin_ref, ck_ref, cv_ref,
                   out_ref, rot_ref, cosb, sinb, sem):
    nb, u, h, d = rot_ref.shape
    i = pl.program_id(0)
    s = pl.program_id(1)

    @pl.when((i == 0) & (s == 0))
    def _prep():
        dmas = []
        for j in range(nb):
            bj = pos_ref[j, 0]
            gc = pltpu.make_async_copy(
                cos_ref.at[pl.ds(bj, u)], cosb.at[j], sem.at[0, j])
            gs = pltpu.make_async_copy(
                sin_ref.at[pl.ds(bj, u)], sinb.at[j], sem.at[1, j])
            gc.start()
            gs.start()
            dmas.append((gc, gs))
        for gc, gs in dmas:
            gc.wait()
            gs.wait()
        x = knew_ref[...]
        xp = pltpu.roll(x, d - 1, 3)
        xm = pltpu.roll(x, 1, 3)
        lane = jax.lax.broadcasted_iota(jnp.int32, x.shape, 3)
        even = (lane % 2) == 0
        half = d // 2
        er = jax.lax.broadcasted_iota(jnp.int32, (half, d), 0)
        ec = jax.lax.broadcasted_iota(jnp.int32, (half, d), 1)
        emat = (ec // 2 == er).astype(jnp.float32)
        cf = jnp.dot(cosb[...].reshape(nb * u, half), emat,
                     preferred_element_type=jnp.float32)
        cf = cf.reshape(nb, u, 1, d)
        sa = jnp.dot(sinb[...].reshape(nb * u, half), emat,
                     preferred_element_type=jnp.float32)
        sa = sa.reshape(nb, u, 1, d)
        sa = jnp.where(even[:, :, :1], -sa, sa)
        rot_ref[...] = x * cf + jnp.where(even, xp, xm) * sa

    out_ref[0, 0] = ck_ref[0]
    out_ref[1, 0] = cv_ref[0]


def _sc_scatter_body(out_ref, rot_ref, vnew_ref, pos_ref,
                     rows_v, idx_v, sem):
    nw = _SC_CORES * _SC_SUBCORES
    nw_half = nw // 2
    total_rows = rot_ref.shape[0]
    rows_per_w = 2 * total_rows // nw
    nb = out_ref.shape[1]
    rows_per_batch = total_rows // nb
    cid = jax.lax.axis_index("c")
    sid = jax.lax.axis_index("s")
    w = sid * _SC_CORES + cid
    kv = w // nw_half
    task = w % nw_half
    roff = task * rows_per_w
    bi = roff // rows_per_batch
    pltpu.sync_copy(pos_ref.at[pl.ds(roff, rows_per_w)], idx_v)

    @pl.when(kv == 0)
    def _stage_k():
        pltpu.sync_copy(rot_ref.at[pl.ds(roff, rows_per_w)], rows_v)

    @pl.when(kv == 1)
    def _stage_v():
        pltpu.sync_copy(vnew_ref.at[pl.ds(roff, rows_per_w)], rows_v)

    cp = pltpu.make_async_copy(rows_v, out_ref.at[kv, bi].at[idx_v], sem)
    cp.start()
    cp.wait()


@functools.partial(jax.jit, static_argnames=("interpret",))
def _run(k_new, v_new, cos, sin, cache_k, cache_v, positions, interpret=False):
    b, u, h, d = k_new.shape
    cl = cache_k.shape[1]
    hd = h * d
    half = d // 2
    f32 = jnp.float32

    t_blk = 512
    s_steps = cl // t_blk
    out1, rotb = pl.pallas_call(
        _copy_rot_body,
        grid=(b, s_steps),
        in_specs=[
            pl.BlockSpec(memory_space=pltpu.SMEM),
            pl.BlockSpec(memory_space=pltpu.VMEM),
            pl.BlockSpec(memory_space=pl.ANY),
            pl.BlockSpec(memory_space=pl.ANY),
            pl.BlockSpec((1, t_blk, h, d), lambda i, s: (i, s, 0, 0)),
            pl.BlockSpec((1, t_blk, h, d), lambda i, s: (i, s, 0, 0)),
        ],
        out_specs=[
            pl.BlockSpec((2, 1, t_blk, h, d), lambda i, s: (0, i, s, 0, 0)),
            pl.BlockSpec((b, u, h, d), lambda i, s: (0, 0, 0, 0)),
        ],
        out_shape=[
            jax.ShapeDtypeStruct((2, b, cl, h, d), f32),
            jax.ShapeDtypeStruct((b, u, h, d), f32),
        ],
        scratch_shapes=[
            pltpu.VMEM((b, u, 1, half), f32),
            pltpu.VMEM((b, u, 1, half), f32),
            pltpu.SemaphoreType.DMA((2, b)),
        ],
        interpret=interpret,
    )(positions, k_new, cos.reshape(cl, 1, half), sin.reshape(cl, 1, half),
      cache_k, cache_v)

    rows_per_w = 2 * b * u // (_SC_CORES * _SC_SUBCORES)
    mesh = plsc.VectorSubcoreMesh(
        core_axis_name="c", subcore_axis_name="s",
        num_cores=_SC_CORES, num_subcores=_SC_SUBCORES)
    sc_scatter = pl.kernel(
        _sc_scatter_body,
        out_type=(),
        mesh=mesh,
        scratch_types=[
            pltpu.VMEM((rows_per_w, h, d), f32),
            pltpu.VMEM((rows_per_w,), jnp.int32),
            pltpu.SemaphoreType.DMA,
        ],
        interpret=interpret,
    )

    oref = jax.new_ref(out1)
    sc_scatter(oref, rotb.reshape(b * u, h, d), v_new.reshape(b * u, h, d),
               positions.reshape(b * u))
    return jax.freeze(oref)


def kernel(k_new, v_new, cos, sin, cache_k, cache_v, positions):
    return _run(k_new, v_new, cos, sin, cache_k, cache_v, positions)
